# Initial kernel scaffold; baseline (speedup 1.0000x reference)
import jax
import jax.numpy as jnp
from jax.experimental import pallas as pl


def kernel(pc, pc_features, params):
    raise NotImplementedError("write your pallas kernel here")


# speedup vs baseline: 14.4867x; 14.4867x over previous
import functools

import jax
import jax.numpy as jnp
import numpy as np
from jax import lax
from jax.experimental import pallas as pl
from jax.experimental.pallas import tpu as pltpu
from jax.experimental.pallas import tpu_sc as plsc

_BN_S = float(1.0 / np.sqrt(1.0 + 1e-5))
_R2_1 = float(0.025 * 0.025)
_R2_2 = float(0.05 * 0.05)
_NEG = -1e30


def _fps_body(px_ref, py_ref, pz_ref, idx_ref, cx_ref, cy_ref, cz_ref, *,
              npoint):
    B, N = px_ref.shape
    px = px_ref[...]
    py = py_ref[...]
    pz = pz_ref[...]
    col = lax.broadcasted_iota(jnp.int32, (B, N), 1).astype(jnp.float32)
    colp = lax.broadcasted_iota(jnp.int32, (B, npoint), 1).astype(jnp.float32)

    def body(i, state):
        far, dists, idx_a, cx_a, cy_a, cz_a = state
        sel = col == far
        cx = jnp.sum(jnp.where(sel, px, 0.0), axis=1, keepdims=True)
        cy = jnp.sum(jnp.where(sel, py, 0.0), axis=1, keepdims=True)
        cz = jnp.sum(jnp.where(sel, pz, 0.0), axis=1, keepdims=True)
        hit = colp == i.astype(jnp.float32)
        idx_a = jnp.where(hit, far, idx_a)
        cx_a = jnp.where(hit, cx, cx_a)
        cy_a = jnp.where(hit, cy, cy_a)
        cz_a = jnp.where(hit, cz, cz_a)
        d = (px - cx) ** 2 + (py - cy) ** 2 + (pz - cz) ** 2
        dists = jnp.minimum(dists, d)
        mx = jnp.max(dists, axis=1, keepdims=True)
        far = jnp.min(jnp.where(dists == mx, col, float(N)),
                      axis=1, keepdims=True)
        return far, dists, idx_a, cx_a, cy_a, cz_a

    zp = px[:, :npoint] * 0.0
    state = (
        px[:, :1] * 0.0,
        px * 0.0 + 1e10,
        zp, zp, zp, zp,
    )
    _, _, idx_a, cx_a, cy_a, cz_a = lax.fori_loop(0, npoint, body, state)
    idx_ref[...] = idx_a.astype(jnp.int32)
    cx_ref[...] = cx_a
    cy_ref[...] = cy_a
    cz_ref[...] = cz_a


def _fps(px, py, pz, npoint):
    B, N = px.shape
    f = jnp.float32
    return pl.pallas_call(
        functools.partial(_fps_body, npoint=npoint),
        out_shape=(
            jax.ShapeDtypeStruct((B, npoint), jnp.int32),
            jax.ShapeDtypeStruct((B, npoint), f),
            jax.ShapeDtypeStruct((B, npoint), f),
            jax.ShapeDtypeStruct((B, npoint), f),
        ),
    )(px, py, pz)


def _bq_body(px_ref, py_ref, pz_ref, cx_ref, cy_ref, cz_ref, out_ref, *,
             r2, nsample):
    b = pl.program_id(0)
    npoint = cx_ref.shape[1]
    N = px_ref.shape[-1]
    px = px_ref[0]
    py = py_ref[0]
    pz = pz_ref[0]
    cx = cx_ref[0]
    cy = cy_ref[0]
    cz = cz_ref[0]
    d2 = (cx - px) ** 2 + (cy - py) ** 2 + (cz - pz) ** 2
    col = lax.broadcasted_iota(jnp.int32, (npoint, N), 1).astype(jnp.float32)
    cols = lax.broadcasted_iota(jnp.int32, (npoint, nsample), 1).astype(jnp.float32)
    fN = float(N)
    cand0 = jnp.where(d2 < r2, col, fN)

    def body(s, state):
        cand, acc = state
        m = jnp.min(cand, axis=1, keepdims=True)
        acc = jnp.where(cols == s.astype(jnp.float32), m, acc)
        cand = jnp.where(cand == m, fN, cand)
        return cand, acc

    _, acc = lax.fori_loop(
        0, nsample, body, (cand0, d2[:, :nsample] * 0.0))
    acc = jnp.where(acc == fN, acc[:, 0:1], acc)
    out_ref[0] = acc.astype(jnp.int32) + b * N


def _bq(px, py, pz, cx, cy, cz, r2, nsample):
    B, N = px.shape
    npoint = cx.shape[1]
    pts = pl.BlockSpec((1, 1, N), lambda b: (b, 0, 0))
    cen = pl.BlockSpec((1, npoint, 1), lambda b: (b, 0, 0))
    return pl.pallas_call(
        functools.partial(_bq_body, r2=r2, nsample=nsample),
        grid=(B,),
        in_specs=[pts, pts, pts, cen, cen, cen],
        out_specs=pl.BlockSpec((1, npoint, nsample), lambda b: (b, 0, 0)),
        out_shape=jax.ShapeDtypeStruct((B, npoint, nsample), jnp.int32),
    )(px.reshape(B, 1, N), py.reshape(B, 1, N), pz.reshape(B, 1, N),
      cx.reshape(B, npoint, 1), cy.reshape(B, npoint, 1),
      cz.reshape(B, npoint, 1))


def _sc_gather(table, idx):
    info = plsc.get_sparse_core_info()
    nw = info.num_cores * info.num_subcores
    R = idx.shape[0]
    chunk = 128
    n_chunks = R // (nw * chunk)
    idx2 = idx.reshape(nw * n_chunks, chunk)
    mesh = plsc.VectorSubcoreMesh(core_axis_name="c", subcore_axis_name="s")

    @functools.partial(
        pl.kernel,
        mesh=mesh,
        compiler_params=pltpu.CompilerParams(use_tc_tiling_on_sc=False),
        out_type=jax.ShapeDtypeStruct((R, 16), jnp.float32),
        scratch_types=[
            pltpu.VMEM((n_chunks, chunk), jnp.int32),
            pltpu.VMEM((chunk, 16), jnp.float32),
            pltpu.SemaphoreType.DMA,
        ],
    )
    def k(table_hbm, idx_hbm, out_hbm, idx_v, row_v, sem):
        wid = lax.axis_index("s") * info.num_cores + lax.axis_index("c")
        base = wid * n_chunks
        pltpu.sync_copy(idx_hbm.at[pl.ds(base, n_chunks)], idx_v)

        def step(j, carry):
            pltpu.async_copy(table_hbm.at[idx_v.at[j]], row_v, sem).wait()
            pltpu.sync_copy(
                row_v, out_hbm.at[pl.ds((base + j) * chunk, chunk)])
            return carry

        lax.fori_loop(0, n_chunks, step, 0)

    return k(table, idx2)


def _mlp1_body(rows_ref, cen_ref, w1_ref, w2_ref, w3_ref, out_ref):
    S = 64
    C = cen_ref.shape[1]
    rows = rows_ref[0]
    cen = cen_ref[0]
    x = (rows.reshape(C, S, 16) - cen[:, None, :]).reshape(C * S, 16)
    x = jnp.maximum(
        jnp.dot(x, w1_ref[...], preferred_element_type=jnp.float32) * _BN_S,
        0.0)
    x = jnp.maximum(
        jnp.dot(x, w2_ref[...], preferred_element_type=jnp.float32) * _BN_S,
        0.0)
    x = jnp.maximum(
        jnp.dot(x, w3_ref[...], preferred_element_type=jnp.float32) * _BN_S,
        0.0)
    out_ref[0] = jnp.max(x.reshape(C, S, 128), axis=1)


def _mlp1(rows, cen, w1, w2, w3):
    B, C = cen.shape[0], cen.shape[1]
    full = lambda s: pl.BlockSpec(s, lambda b: (0, 0))
    return pl.pallas_call(
        _mlp1_body,
        grid=(B,),
        in_specs=[
            pl.BlockSpec((1, C * 64, 16), lambda b: (b, 0, 0)),
            pl.BlockSpec((1, C, 16), lambda b: (b, 0, 0)),
            full(w1.shape), full(w2.shape), full(w3.shape),
        ],
        out_specs=pl.BlockSpec((1, C, 128), lambda b: (b, 0, 0)),
        out_shape=jax.ShapeDtypeStruct((B, C, 128), jnp.float32),
    )(rows, cen, w1, w2, w3)


def _sa2_body(f_ref, pxc_ref, pyc_ref, pzc_ref, pxr_ref, pyr_ref, pzr_ref,
              c2x_ref, c2y_ref, c2z_ref, w1f_ref, w1p_ref, w2_ref, w3_ref,
              out_ref):
    NP = f_ref.shape[1]
    M = c2x_ref.shape[1]
    F = f_ref[0]
    pxc, pyc, pzc = pxc_ref[0], pyc_ref[0], pzc_ref[0]
    pxr, pyr, pzr = pxr_ref[0], pyr_ref[0], pzr_ref[0]
    c2x, c2y, c2z = c2x_ref[0], c2y_ref[0], c2z_ref[0]
    w1x = w1p_ref[0:1, :]
    w1y = w1p_ref[1:2, :]
    w1z = w1p_ref[2:3, :]
    A = jnp.dot(F, w1f_ref[...], preferred_element_type=jnp.float32)
    A = A + pxc * w1x + pyc * w1y + pzc * w1z
    Bc = -(c2x * w1x + c2y * w1y + c2z * w1z)
    x = jnp.maximum((A[None, :, :] + Bc[:, None, :]) * _BN_S, 0.0)
    x = x.reshape(M * NP, 128)
    x = jnp.maximum(
        jnp.dot(x, w2_ref[...], preferred_element_type=jnp.float32) * _BN_S,
        0.0)
    x = jnp.maximum(
        jnp.dot(x, w3_ref[...], preferred_element_type=jnp.float32) * _BN_S,
        0.0)
    d2 = (c2x - pxr) ** 2 + (c2y - pyr) ** 2 + (c2z - pzr) ** 2
    pen = jnp.where(d2 < _R2_2, 0.0, _NEG)
    x = x.reshape(M, NP, 256) + pen[:, :, None]
    out_ref[0] = jnp.max(x, axis=1)


def _sa2(feat1, pxc, pyc, pzc, pxr, pyr, pzr, c2x, c2y, c2z,
         w1f, w1p, w2, w3):
    B, NP = feat1.shape[0], feat1.shape[1]
    M = c2x.shape[1]
    colc = pl.BlockSpec((1, NP, 1), lambda b: (b, 0, 0))
    rowc = pl.BlockSpec((1, 1, NP), lambda b: (b, 0, 0))
    cen = pl.BlockSpec((1, M, 1), lambda b: (b, 0, 0))
    full = lambda s: pl.BlockSpec(s, lambda b: (0, 0))
    return pl.pallas_call(
        _sa2_body,
        grid=(B,),
        in_specs=[
            pl.BlockSpec((1, NP, 128), lambda b: (b, 0, 0)),
            colc, colc, colc, rowc, rowc, rowc, cen, cen, cen,
            full(w1f.shape), full(w1p.shape), full(w2.shape), full(w3.shape),
        ],
        out_specs=pl.BlockSpec((1, M, 256), lambda b: (b, 0, 0)),
        out_shape=jax.ShapeDtypeStruct((B, M, 256), jnp.float32),
    )(feat1, pxc, pyc, pzc, pxr, pyr, pzr, c2x, c2y, c2z, w1f, w1p, w2, w3)


def _sa3fc_body(f_ref, cx_ref, cy_ref, cz_ref, w1f_ref, w1p_ref, w2_ref,
                w3_ref, wf1_ref, b1_ref, wf2_ref, b2_ref, wf3_ref, b3_ref,
                out_ref):
    F = f_ref[0]
    cx, cy, cz = cx_ref[0], cy_ref[0], cz_ref[0]
    w1x = w1p_ref[0:1, :]
    w1y = w1p_ref[1:2, :]
    w1z = w1p_ref[2:3, :]
    A = jnp.dot(F, w1f_ref[...], preferred_element_type=jnp.float32)
    A = A + cx * w1x + cy * w1y + cz * w1z
    x = jnp.maximum(A * _BN_S, 0.0)
    x = jnp.maximum(
        jnp.dot(x, w2_ref[...], preferred_element_type=jnp.float32) * _BN_S,
        0.0)
    x = jnp.maximum(
        jnp.dot(x, w3_ref[...], preferred_element_type=jnp.float32) * _BN_S,
        0.0)
    g = jnp.max(x, axis=0, keepdims=True)
    h = jnp.maximum(
        (jnp.dot(g, wf1_ref[...], preferred_element_type=jnp.float32)
         + b1_ref[...]) * _BN_S, 0.0)
    h = jnp.maximum(
        (jnp.dot(h, wf2_ref[...], preferred_element_type=jnp.float32)
         + b2_ref[...]) * _BN_S, 0.0)
    o = (jnp.dot(h, wf3_ref[...], preferred_element_type=jnp.float32)
         + b3_ref[...])
    out_ref[0] = 1.0 / (1.0 + jnp.exp(-o))


def _sa3fc(feat2, cx, cy, cz, w1f, w1p, w2, w3, wf1, b1, wf2, b2, wf3, b3):
    B, M = feat2.shape[0], feat2.shape[1]
    cen = pl.BlockSpec((1, M, 1), lambda b: (b, 0, 0))
    full = lambda s: pl.BlockSpec(s, lambda b: (0, 0))
    out = pl.pallas_call(
        _sa3fc_body,
        grid=(B,),
        in_specs=[
            pl.BlockSpec((1, M, 256), lambda b: (b, 0, 0)),
            cen, cen, cen,
            full(w1f.shape), full(w1p.shape), full(w2.shape), full(w3.shape),
            full(wf1.shape), full(b1.shape), full(wf2.shape), full(b2.shape),
            full(wf3.shape), full(b3.shape),
        ],
        out_specs=pl.BlockSpec((1, 1, 1), lambda b: (b, 0, 0)),
        out_shape=jax.ShapeDtypeStruct((B, 1, 1), jnp.float32),
    )(feat2, cx, cy, cz, w1f, w1p, w2, w3, wf1, b1, wf2, b2, wf3, b3)
    return out.reshape(B, 1)


def _pad_xyz_rows(w, width):
    t = w[:, :3].T
    return jnp.concatenate(
        [t, jnp.zeros((8 - 3, width), jnp.float32)], axis=0)


def kernel(pc, pc_features, params):
    B, N, _ = pc.shape
    f32 = jnp.float32
    px, py, pz = pc[:, :, 0], pc[:, :, 1], pc[:, :, 2]

    idx1, cx1, cy1, cz1 = _fps(px, py, pz, 128)
    nidx = _bq(px, py, pz, cx1, cy1, cz1, _R2_1, 64)

    feats = jnp.transpose(pc_features, (0, 2, 1))
    table = jnp.concatenate(
        [pc, feats, jnp.zeros((B, N, 9), f32)], axis=-1).reshape(B * N, 16)
    rows = _sc_gather(table, nidx.reshape(-1))

    cen1 = jnp.concatenate(
        [cx1[..., None], cy1[..., None], cz1[..., None],
         jnp.zeros((B, 128, 13), f32)], axis=-1)
    w1a, w2a, w3a = params['sa1']
    w1p = jnp.concatenate(
        [w1a.T, jnp.zeros((16 - 7, 64), f32)], axis=0)
    feat1 = _mlp1(rows.reshape(B, 128 * 64, 16), cen1, w1p, w2a.T, w3a.T)

    idx2, cx2, cy2, cz2 = _fps(cx1, cy1, cz1, 32)
    w1b, w2b, w3b = params['sa2']
    feat2 = _sa2(
        feat1,
        cx1.reshape(B, 128, 1), cy1.reshape(B, 128, 1), cz1.reshape(B, 128, 1),
        cx1.reshape(B, 1, 128), cy1.reshape(B, 1, 128), cz1.reshape(B, 1, 128),
        cx2.reshape(B, 32, 1), cy2.reshape(B, 32, 1), cz2.reshape(B, 32, 1),
        w1b[:, 3:].T, _pad_xyz_rows(w1b, 128), w2b.T, w3b.T)

    w1c, w2c, w3c = params['sa3']
    wf1, b1, wf2, b2, wf3, b3 = params['fc']
    return _sa3fc(
        feat2,
        cx2.reshape(B, 32, 1), cy2.reshape(B, 32, 1), cz2.reshape(B, 32, 1),
        w1c[:, 3:].T, _pad_xyz_rows(w1c, 256), w2c.T, w3c.T,
        wf1.T, b1.reshape(1, 1024), wf2.T, b2.reshape(1, 1024),
        wf3.T, b3.reshape(1, 1))


# speedup vs baseline: 36.5233x; 2.5212x over previous
import functools

import jax
import jax.numpy as jnp
import numpy as np
from jax import lax
from jax.experimental import pallas as pl
from jax.experimental.pallas import tpu as pltpu
from jax.experimental.pallas import tpu_sc as plsc

_BN_S = float(1.0 / np.sqrt(1.0 + 1e-5))
_R2_1 = float(0.025 * 0.025)
_R2_2 = float(0.05 * 0.05)
_NEG = -1e30


def _fps_body(px_ref, py_ref, pz_ref, idx_ref, cx_ref, cy_ref, cz_ref, *,
              npoint):
    B, N = px_ref.shape
    px = px_ref[...]
    py = py_ref[...]
    pz = pz_ref[...]
    col = lax.broadcasted_iota(jnp.int32, (B, N), 1).astype(jnp.float32)
    colp = lax.broadcasted_iota(jnp.int32, (B, npoint), 1).astype(jnp.float32)

    def body(i, state):
        far, dists, idx_a, cx_a, cy_a, cz_a = state
        sel = col == far
        cx = jnp.sum(jnp.where(sel, px, 0.0), axis=1, keepdims=True)
        cy = jnp.sum(jnp.where(sel, py, 0.0), axis=1, keepdims=True)
        cz = jnp.sum(jnp.where(sel, pz, 0.0), axis=1, keepdims=True)
        hit = colp == i.astype(jnp.float32)
        idx_a = jnp.where(hit, far, idx_a)
        cx_a = jnp.where(hit, cx, cx_a)
        cy_a = jnp.where(hit, cy, cy_a)
        cz_a = jnp.where(hit, cz, cz_a)
        d = (px - cx) ** 2 + (py - cy) ** 2 + (pz - cz) ** 2
        dists = jnp.minimum(dists, d)
        mx = jnp.max(dists, axis=1, keepdims=True)
        far = jnp.min(jnp.where(dists == mx, col, float(N)),
                      axis=1, keepdims=True)
        return far, dists, idx_a, cx_a, cy_a, cz_a

    zp = px[:, :npoint] * 0.0
    state = (
        px[:, :1] * 0.0,
        px * 0.0 + 1e10,
        zp, zp, zp, zp,
    )
    _, _, idx_a, cx_a, cy_a, cz_a = lax.fori_loop(0, npoint, body, state)
    idx_ref[...] = idx_a.astype(jnp.int32)
    cx_ref[...] = cx_a
    cy_ref[...] = cy_a
    cz_ref[...] = cz_a


def _fps(px, py, pz, npoint):
    B, N = px.shape
    f = jnp.float32
    return pl.pallas_call(
        functools.partial(_fps_body, npoint=npoint),
        out_shape=(
            jax.ShapeDtypeStruct((B, npoint), jnp.int32),
            jax.ShapeDtypeStruct((B, npoint), f),
            jax.ShapeDtypeStruct((B, npoint), f),
            jax.ShapeDtypeStruct((B, npoint), f),
        ),
    )(px, py, pz)


def _bq_body(px_ref, py_ref, pz_ref, cx_ref, cy_ref, cz_ref, out_ref, *,
             r2, nsample):
    b = pl.program_id(0)
    npoint = cx_ref.shape[1]
    N = px_ref.shape[-1]
    px = px_ref[0]
    py = py_ref[0]
    pz = pz_ref[0]
    cx = cx_ref[0]
    cy = cy_ref[0]
    cz = cz_ref[0]
    d2 = (cx - px) ** 2 + (cy - py) ** 2 + (cz - pz) ** 2
    col = lax.broadcasted_iota(jnp.int32, (npoint, N), 1).astype(jnp.float32)
    cols = lax.broadcasted_iota(jnp.int32, (npoint, nsample), 1).astype(jnp.float32)
    fN = float(N)
    within = d2 < r2
    cand0 = jnp.where(within, col, fN)
    cnt = jnp.sum(jnp.where(within, 1.0, 0.0), axis=1, keepdims=True)
    cap = jnp.minimum(jnp.max(cnt), float(nsample)).astype(jnp.int32)

    def cond(state):
        s, _, _ = state
        return s < cap

    def body(state):
        s, cand, acc = state
        m = jnp.min(cand, axis=1, keepdims=True)
        acc = jnp.where(cols == s.astype(jnp.float32), m, acc)
        cand = jnp.where(cand == m, fN, cand)
        return s + 1, cand, acc

    _, _, acc = lax.while_loop(
        cond, body, (jnp.int32(0), cand0, d2[:, :nsample] * 0.0 + fN))
    acc = jnp.where(acc == fN, acc[:, 0:1], acc)
    out_ref[0] = acc.astype(jnp.int32) + b * N


def _bq(px, py, pz, cx, cy, cz, r2, nsample):
    B, N = px.shape
    npoint = cx.shape[1]
    pts = pl.BlockSpec((1, 1, N), lambda b: (b, 0, 0))
    cen = pl.BlockSpec((1, npoint, 1), lambda b: (b, 0, 0))
    return pl.pallas_call(
        functools.partial(_bq_body, r2=r2, nsample=nsample),
        grid=(B,),
        in_specs=[pts, pts, pts, cen, cen, cen],
        out_specs=pl.BlockSpec((1, npoint, nsample), lambda b: (b, 0, 0)),
        out_shape=jax.ShapeDtypeStruct((B, npoint, nsample), jnp.int32),
    )(px.reshape(B, 1, N), py.reshape(B, 1, N), pz.reshape(B, 1, N),
      cx.reshape(B, npoint, 1), cy.reshape(B, npoint, 1),
      cz.reshape(B, npoint, 1))


def _sc_gather(table, idx):
    info = plsc.get_sparse_core_info()
    nw = info.num_cores * info.num_subcores
    R = idx.shape[0]
    chunk = 128
    n_chunks = R // (nw * chunk)
    idx2 = idx.reshape(nw * n_chunks, chunk)
    mesh = plsc.VectorSubcoreMesh(core_axis_name="c", subcore_axis_name="s")

    @functools.partial(
        pl.kernel,
        mesh=mesh,
        compiler_params=pltpu.CompilerParams(use_tc_tiling_on_sc=False),
        out_type=jax.ShapeDtypeStruct((nw * n_chunks, chunk, 16),
                                      jnp.float32),
        scratch_types=[
            pltpu.VMEM((n_chunks, chunk), jnp.int32),
            pltpu.VMEM((n_chunks, chunk, 16), jnp.float32),
            pltpu.SemaphoreType.DMA,
        ],
    )
    def k(table_hbm, idx_hbm, out_hbm, idx_v, rows_v, sem):
        wid = lax.axis_index("s") * info.num_cores + lax.axis_index("c")
        base = wid * n_chunks
        pltpu.sync_copy(idx_hbm.at[pl.ds(base, n_chunks)], idx_v)

        def fire(j, carry):
            pltpu.async_copy(table_hbm.at[idx_v.at[j]], rows_v.at[j], sem)
            return carry

        lax.fori_loop(0, n_chunks, fire, 0)

        def drain(j, carry):
            pltpu.make_async_copy(
                table_hbm.at[idx_v.at[j]], rows_v.at[j], sem).wait()
            return carry

        lax.fori_loop(0, n_chunks, drain, 0)
        pltpu.sync_copy(rows_v, out_hbm.at[pl.ds(base, n_chunks)])

    return k(table, idx2).reshape(R, 16)


def _mlp1_body(rows_ref, cen_ref, w1_ref, w2_ref, w3_ref, out_ref):
    S = 64
    C = cen_ref.shape[1]
    rows = rows_ref[0]
    cen = cen_ref[0]
    x = (rows.reshape(C, S, 16) - cen[:, None, :]).reshape(C * S, 16)
    x = jnp.maximum(
        jnp.dot(x, w1_ref[...], preferred_element_type=jnp.float32) * _BN_S,
        0.0)
    x = jnp.maximum(
        jnp.dot(x, w2_ref[...], preferred_element_type=jnp.float32) * _BN_S,
        0.0)
    x = jnp.maximum(
        jnp.dot(x, w3_ref[...], preferred_element_type=jnp.float32) * _BN_S,
        0.0)
    out_ref[0] = jnp.max(x.reshape(C, S, 128), axis=1)


def _mlp1(rows, cen, w1, w2, w3):
    B, C = cen.shape[0], cen.shape[1]
    full = lambda s: pl.BlockSpec(s, lambda b: (0, 0))
    return pl.pallas_call(
        _mlp1_body,
        grid=(B,),
        in_specs=[
            pl.BlockSpec((1, C * 64, 16), lambda b: (b, 0, 0)),
            pl.BlockSpec((1, C, 16), lambda b: (b, 0, 0)),
            full(w1.shape), full(w2.shape), full(w3.shape),
        ],
        out_specs=pl.BlockSpec((1, C, 128), lambda b: (b, 0, 0)),
        out_shape=jax.ShapeDtypeStruct((B, C, 128), jnp.float32),
    )(rows, cen, w1, w2, w3)


def _sa2_body(f_ref, pxc_ref, pyc_ref, pzc_ref, pxr_ref, pyr_ref, pzr_ref,
              c2x_ref, c2y_ref, c2z_ref, w1f_ref, w1p_ref, w2_ref, w3_ref,
              out_ref):
    NP = f_ref.shape[1]
    M = c2x_ref.shape[1]
    F = f_ref[0]
    pxc, pyc, pzc = pxc_ref[0], pyc_ref[0], pzc_ref[0]
    pxr, pyr, pzr = pxr_ref[0], pyr_ref[0], pzr_ref[0]
    c2x, c2y, c2z = c2x_ref[0], c2y_ref[0], c2z_ref[0]
    w1x = w1p_ref[0:1, :]
    w1y = w1p_ref[1:2, :]
    w1z = w1p_ref[2:3, :]
    A = jnp.dot(F, w1f_ref[...], preferred_element_type=jnp.float32)
    A = A + pxc * w1x + pyc * w1y + pzc * w1z
    Bc = -(c2x * w1x + c2y * w1y + c2z * w1z)
    x = jnp.maximum((A[None, :, :] + Bc[:, None, :]) * _BN_S, 0.0)
    x = x.reshape(M * NP, 128)
    x = jnp.maximum(
        jnp.dot(x, w2_ref[...], preferred_element_type=jnp.float32) * _BN_S,
        0.0)
    x = jnp.maximum(
        jnp.dot(x, w3_ref[...], preferred_element_type=jnp.float32) * _BN_S,
        0.0)
    d2 = (c2x - pxr) ** 2 + (c2y - pyr) ** 2 + (c2z - pzr) ** 2
    pen = jnp.where(d2 < _R2_2, 0.0, _NEG)
    x = x.reshape(M, NP, 256) + pen[:, :, None]
    out_ref[0] = jnp.max(x, axis=1)


def _sa2(feat1, pxc, pyc, pzc, pxr, pyr, pzr, c2x, c2y, c2z,
         w1f, w1p, w2, w3):
    B, NP = feat1.shape[0], feat1.shape[1]
    M = c2x.shape[1]
    colc = pl.BlockSpec((1, NP, 1), lambda b: (b, 0, 0))
    rowc = pl.BlockSpec((1, 1, NP), lambda b: (b, 0, 0))
    cen = pl.BlockSpec((1, M, 1), lambda b: (b, 0, 0))
    full = lambda s: pl.BlockSpec(s, lambda b: (0, 0))
    return pl.pallas_call(
        _sa2_body,
        grid=(B,),
        in_specs=[
            pl.BlockSpec((1, NP, 128), lambda b: (b, 0, 0)),
            colc, colc, colc, rowc, rowc, rowc, cen, cen, cen,
            full(w1f.shape), full(w1p.shape), full(w2.shape), full(w3.shape),
        ],
        out_specs=pl.BlockSpec((1, M, 256), lambda b: (b, 0, 0)),
        out_shape=jax.ShapeDtypeStruct((B, M, 256), jnp.float32),
    )(feat1, pxc, pyc, pzc, pxr, pyr, pzr, c2x, c2y, c2z, w1f, w1p, w2, w3)


def _sa3fc_body(f_ref, cx_ref, cy_ref, cz_ref, w1f_ref, w1p_ref, w2_ref,
                w3_ref, wf1_ref, b1_ref, wf2_ref, b2_ref, wf3_ref, b3_ref,
                out_ref):
    F = f_ref[0]
    cx, cy, cz = cx_ref[0], cy_ref[0], cz_ref[0]
    w1x = w1p_ref[0:1, :]
    w1y = w1p_ref[1:2, :]
    w1z = w1p_ref[2:3, :]
    A = jnp.dot(F, w1f_ref[...], preferred_element_type=jnp.float32)
    A = A + cx * w1x + cy * w1y + cz * w1z
    x = jnp.maximum(A * _BN_S, 0.0)
    x = jnp.maximum(
        jnp.dot(x, w2_ref[...], preferred_element_type=jnp.float32) * _BN_S,
        0.0)
    x = jnp.maximum(
        jnp.dot(x, w3_ref[...], preferred_element_type=jnp.float32) * _BN_S,
        0.0)
    g = jnp.max(x, axis=0, keepdims=True)
    h = jnp.maximum(
        (jnp.dot(g, wf1_ref[...], preferred_element_type=jnp.float32)
         + b1_ref[...]) * _BN_S, 0.0)
    h = jnp.maximum(
        (jnp.dot(h, wf2_ref[...], preferred_element_type=jnp.float32)
         + b2_ref[...]) * _BN_S, 0.0)
    o = (jnp.dot(h, wf3_ref[...], preferred_element_type=jnp.float32)
         + b3_ref[...])
    out_ref[0] = 1.0 / (1.0 + jnp.exp(-o))


def _sa3fc(feat2, cx, cy, cz, w1f, w1p, w2, w3, wf1, b1, wf2, b2, wf3, b3):
    B, M = feat2.shape[0], feat2.shape[1]
    cen = pl.BlockSpec((1, M, 1), lambda b: (b, 0, 0))
    full = lambda s: pl.BlockSpec(s, lambda b: (0, 0))
    out = pl.pallas_call(
        _sa3fc_body,
        grid=(B,),
        in_specs=[
            pl.BlockSpec((1, M, 256), lambda b: (b, 0, 0)),
            cen, cen, cen,
            full(w1f.shape), full(w1p.shape), full(w2.shape), full(w3.shape),
            full(wf1.shape), full(b1.shape), full(wf2.shape), full(b2.shape),
            full(wf3.shape), full(b3.shape),
        ],
        out_specs=pl.BlockSpec((1, 1, 1), lambda b: (b, 0, 0)),
        out_shape=jax.ShapeDtypeStruct((B, 1, 1), jnp.float32),
    )(feat2, cx, cy, cz, w1f, w1p, w2, w3, wf1, b1, wf2, b2, wf3, b3)
    return out.reshape(B, 1)


def _pad_xyz_rows(w, width):
    t = w[:, :3].T
    return jnp.concatenate(
        [t, jnp.zeros((8 - 3, width), jnp.float32)], axis=0)


def kernel(pc, pc_features, params):
    B, N, _ = pc.shape
    f32 = jnp.float32
    px, py, pz = pc[:, :, 0], pc[:, :, 1], pc[:, :, 2]

    idx1, cx1, cy1, cz1 = _fps(px, py, pz, 128)
    nidx = _bq(px, py, pz, cx1, cy1, cz1, _R2_1, 64)

    feats = jnp.transpose(pc_features, (0, 2, 1))
    table = jnp.concatenate(
        [pc, feats, jnp.zeros((B, N, 9), f32)], axis=-1).reshape(B * N, 16)
    rows = _sc_gather(table, nidx.reshape(-1))

    cen1 = jnp.concatenate(
        [cx1[..., None], cy1[..., None], cz1[..., None],
         jnp.zeros((B, 128, 13), f32)], axis=-1)
    w1a, w2a, w3a = params['sa1']
    w1p = jnp.concatenate(
        [w1a.T, jnp.zeros((16 - 7, 64), f32)], axis=0)
    feat1 = _mlp1(rows.reshape(B, 128 * 64, 16), cen1, w1p, w2a.T, w3a.T)

    idx2, cx2, cy2, cz2 = _fps(cx1, cy1, cz1, 32)
    w1b, w2b, w3b = params['sa2']
    feat2 = _sa2(
        feat1,
        cx1.reshape(B, 128, 1), cy1.reshape(B, 128, 1), cz1.reshape(B, 128, 1),
        cx1.reshape(B, 1, 128), cy1.reshape(B, 1, 128), cz1.reshape(B, 1, 128),
        cx2.reshape(B, 32, 1), cy2.reshape(B, 32, 1), cz2.reshape(B, 32, 1),
        w1b[:, 3:].T, _pad_xyz_rows(w1b, 128), w2b.T, w3b.T)

    w1c, w2c, w3c = params['sa3']
    wf1, b1, wf2, b2, wf3, b3 = params['fc']
    return _sa3fc(
        feat2,
        cx2.reshape(B, 32, 1), cy2.reshape(B, 32, 1), cz2.reshape(B, 32, 1),
        w1c[:, 3:].T, _pad_xyz_rows(w1c, 256), w2c.T, w3c.T,
        wf1.T, b1.reshape(1, 1024), wf2.T, b2.reshape(1, 1024),
        wf3.T, b3.reshape(1, 1))


# speedup vs baseline: 40.3445x; 1.1046x over previous
import functools

import jax
import jax.numpy as jnp
import numpy as np
from jax import lax
from jax.experimental import pallas as pl
from jax.experimental.pallas import tpu as pltpu
from jax.experimental.pallas import tpu_sc as plsc

_BN_S = float(1.0 / np.sqrt(1.0 + 1e-5))
_R2_1 = float(0.025 * 0.025)
_R2_2 = float(0.05 * 0.05)
_NEG = -1e30


def _fps_body(px_ref, py_ref, pz_ref, idx_ref, cx_ref, cy_ref, cz_ref, *,
              npoint):
    B, N = px_ref.shape
    px = px_ref[...]
    py = py_ref[...]
    pz = pz_ref[...]
    col = lax.broadcasted_iota(jnp.int32, (B, N), 1).astype(jnp.float32)
    colp = lax.broadcasted_iota(jnp.int32, (B, npoint), 1).astype(jnp.float32)

    def body(i, state):
        far, dists, idx_a, cx_a, cy_a, cz_a = state
        sel = col == far
        cx = jnp.sum(jnp.where(sel, px, 0.0), axis=1, keepdims=True)
        cy = jnp.sum(jnp.where(sel, py, 0.0), axis=1, keepdims=True)
        cz = jnp.sum(jnp.where(sel, pz, 0.0), axis=1, keepdims=True)
        hit = colp == i.astype(jnp.float32)
        idx_a = jnp.where(hit, far, idx_a)
        cx_a = jnp.where(hit, cx, cx_a)
        cy_a = jnp.where(hit, cy, cy_a)
        cz_a = jnp.where(hit, cz, cz_a)
        d = (px - cx) ** 2 + (py - cy) ** 2 + (pz - cz) ** 2
        dists = jnp.minimum(dists, d)
        mx = jnp.max(dists, axis=1, keepdims=True)
        far = jnp.min(jnp.where(dists == mx, col, float(N)),
                      axis=1, keepdims=True)
        return far, dists, idx_a, cx_a, cy_a, cz_a

    zp = px[:, :npoint] * 0.0
    state = (
        px[:, :1] * 0.0,
        px * 0.0 + 1e10,
        zp, zp, zp, zp,
    )
    _, _, idx_a, cx_a, cy_a, cz_a = lax.fori_loop(0, npoint, body, state)
    idx_ref[...] = idx_a.astype(jnp.int32)
    cx_ref[...] = cx_a
    cy_ref[...] = cy_a
    cz_ref[...] = cz_a


def _fps(px, py, pz, npoint):
    B, N = px.shape
    f = jnp.float32
    return pl.pallas_call(
        functools.partial(_fps_body, npoint=npoint),
        out_shape=(
            jax.ShapeDtypeStruct((B, npoint), jnp.int32),
            jax.ShapeDtypeStruct((B, npoint), f),
            jax.ShapeDtypeStruct((B, npoint), f),
            jax.ShapeDtypeStruct((B, npoint), f),
        ),
    )(px, py, pz)


def _bq_body(px_ref, py_ref, pz_ref, cx_ref, cy_ref, cz_ref, out_ref, *,
             r2, nsample):
    b = pl.program_id(0)
    npoint = cx_ref.shape[1]
    N = px_ref.shape[-1]
    px = px_ref[0]
    py = py_ref[0]
    pz = pz_ref[0]
    cx = cx_ref[0]
    cy = cy_ref[0]
    cz = cz_ref[0]
    d2 = (cx - px) ** 2 + (cy - py) ** 2 + (cz - pz) ** 2
    col = lax.broadcasted_iota(jnp.int32, (npoint, N), 1).astype(jnp.float32)
    cols = lax.broadcasted_iota(jnp.int32, (npoint, nsample), 1).astype(jnp.float32)
    fN = float(N)
    within = d2 < r2
    cand0 = jnp.where(within, col, fN)
    cnt = jnp.sum(jnp.where(within, 1.0, 0.0), axis=1, keepdims=True)
    cap = jnp.minimum(jnp.max(cnt), float(nsample)).astype(jnp.int32)

    def cond(state):
        s, _, _ = state
        return s < cap

    def body(state):
        s, cand, acc = state
        m = jnp.min(cand, axis=1, keepdims=True)
        acc = jnp.where(cols == s.astype(jnp.float32), m, acc)
        cand = jnp.where(cand == m, fN, cand)
        return s + 1, cand, acc

    _, _, acc = lax.while_loop(
        cond, body, (jnp.int32(0), cand0, d2[:, :nsample] * 0.0 + fN))
    acc = jnp.where(acc == fN, acc[:, 0:1], acc)
    out_ref[0] = acc.astype(jnp.int32) + b * N


def _bq(px, py, pz, cx, cy, cz, r2, nsample):
    B, N = px.shape
    npoint = cx.shape[1]
    pts = pl.BlockSpec((1, 1, N), lambda b: (b, 0, 0))
    cen = pl.BlockSpec((1, npoint, 1), lambda b: (b, 0, 0))
    return pl.pallas_call(
        functools.partial(_bq_body, r2=r2, nsample=nsample),
        grid=(B,),
        in_specs=[pts, pts, pts, cen, cen, cen],
        out_specs=pl.BlockSpec((1, npoint, nsample), lambda b: (b, 0, 0)),
        out_shape=jax.ShapeDtypeStruct((B, npoint, nsample), jnp.int32),
    )(px.reshape(B, 1, N), py.reshape(B, 1, N), pz.reshape(B, 1, N),
      cx.reshape(B, npoint, 1), cy.reshape(B, npoint, 1),
      cz.reshape(B, npoint, 1))


def _sc_gather(table, idx):
    info = plsc.get_sparse_core_info()
    nw = info.num_cores * info.num_subcores
    R = idx.shape[0]
    chunk = 128
    n_chunks = R // (nw * chunk)
    idx2 = idx.reshape(nw * n_chunks, chunk)
    mesh = plsc.VectorSubcoreMesh(core_axis_name="c", subcore_axis_name="s")

    @functools.partial(
        pl.kernel,
        mesh=mesh,
        compiler_params=pltpu.CompilerParams(use_tc_tiling_on_sc=False),
        out_type=jax.ShapeDtypeStruct((nw * n_chunks, chunk, 16),
                                      jnp.float32),
        scratch_types=[
            pltpu.VMEM((n_chunks, chunk), jnp.int32),
            pltpu.VMEM((n_chunks, chunk, 16), jnp.float32),
            pltpu.SemaphoreType.DMA,
        ],
    )
    def k(table_hbm, idx_hbm, out_hbm, idx_v, rows_v, sem):
        wid = lax.axis_index("s") * info.num_cores + lax.axis_index("c")
        base = wid * n_chunks
        pltpu.sync_copy(idx_hbm.at[pl.ds(base, n_chunks)], idx_v)

        def fire(j, carry):
            pltpu.async_copy(table_hbm.at[idx_v.at[j]], rows_v.at[j], sem)
            return carry

        lax.fori_loop(0, n_chunks, fire, 0)

        def drain(j, carry):
            pltpu.make_async_copy(
                table_hbm.at[idx_v.at[j]], rows_v.at[j], sem).wait()
            return carry

        lax.fori_loop(0, n_chunks, drain, 0)
        pltpu.sync_copy(rows_v, out_hbm.at[pl.ds(base, n_chunks)])

    return k(table, idx2).reshape(R, 16)


def _mlp1_body(rows_ref, cen_ref, w1_ref, w1bd_ref, w2_ref, w3_ref,
               out_ref):
    C = cen_ref.shape[1]
    x = rows_ref[0]
    cen = cen_ref[0]
    y = jnp.dot(x, w1bd_ref[...],
                preferred_element_type=jnp.float32)
    y = y.reshape(C * 32, 128)
    bc = jnp.dot(cen, w1_ref[...],
                 preferred_element_type=jnp.float32)
    bc = jnp.broadcast_to(bc[:, None, :], (C, 32, 64)).reshape(C * 32, 64)
    bc = jnp.concatenate([bc, bc], axis=1)
    y = jnp.maximum((y - bc) * _BN_S, 0.0)
    y = jnp.maximum(
        jnp.dot(y, w2_ref[...], preferred_element_type=jnp.float32) * _BN_S,
        0.0)
    y = jnp.maximum(
        jnp.dot(y, w3_ref[...], preferred_element_type=jnp.float32) * _BN_S,
        0.0)
    y = y.reshape(C * 64, 128)
    out_ref[0] = jnp.max(y.reshape(C, 64, 128), axis=1)


def _mlp1(rows, cen, w1, w1bd, w2bd, w3bd):
    B, C = cen.shape[0], cen.shape[1]
    full = lambda s: pl.BlockSpec(s, lambda b: (0, 0))
    return pl.pallas_call(
        _mlp1_body,
        grid=(B,),
        in_specs=[
            pl.BlockSpec((1, C * 8, 128), lambda b: (b, 0, 0)),
            pl.BlockSpec((1, C, 16), lambda b: (b, 0, 0)),
            full(w1.shape), full(w1bd.shape), full(w2bd.shape),
            full(w3bd.shape),
        ],
        out_specs=pl.BlockSpec((1, C, 128), lambda b: (b, 0, 0)),
        out_shape=jax.ShapeDtypeStruct((B, C, 128), jnp.float32),
    )(rows, cen, w1, w1bd, w2bd, w3bd)


def _sa2_body(f_ref, pxc_ref, pyc_ref, pzc_ref, pxr_ref, pyr_ref, pzr_ref,
              c2x_ref, c2y_ref, c2z_ref, w1f_ref, w1p_ref, w2_ref, w3_ref,
              out_ref):
    NP = f_ref.shape[1]
    M = c2x_ref.shape[1]
    F = f_ref[0]
    pxc, pyc, pzc = pxc_ref[0], pyc_ref[0], pzc_ref[0]
    pxr, pyr, pzr = pxr_ref[0], pyr_ref[0], pzr_ref[0]
    c2x, c2y, c2z = c2x_ref[0], c2y_ref[0], c2z_ref[0]
    w1x = w1p_ref[0:1, :]
    w1y = w1p_ref[1:2, :]
    w1z = w1p_ref[2:3, :]
    A = jnp.dot(F, w1f_ref[...], preferred_element_type=jnp.float32)
    A = A + pxc * w1x + pyc * w1y + pzc * w1z
    Bc = -(c2x * w1x + c2y * w1y + c2z * w1z)
    x = jnp.maximum((A[None, :, :] + Bc[:, None, :]) * _BN_S, 0.0)
    x = x.reshape(M * NP, 128)
    x = jnp.maximum(
        jnp.dot(x, w2_ref[...], preferred_element_type=jnp.float32) * _BN_S,
        0.0)
    x = jnp.maximum(
        jnp.dot(x, w3_ref[...], preferred_element_type=jnp.float32) * _BN_S,
        0.0)
    d2 = (c2x - pxr) ** 2 + (c2y - pyr) ** 2 + (c2z - pzr) ** 2
    pen = jnp.where(d2 < _R2_2, 0.0, _NEG)
    x = x.reshape(M, NP, 256) + pen[:, :, None]
    out_ref[0] = jnp.max(x, axis=1)


def _sa2(feat1, pxc, pyc, pzc, pxr, pyr, pzr, c2x, c2y, c2z,
         w1f, w1p, w2, w3):
    B, NP = feat1.shape[0], feat1.shape[1]
    M = c2x.shape[1]
    colc = pl.BlockSpec((1, NP, 1), lambda b: (b, 0, 0))
    rowc = pl.BlockSpec((1, 1, NP), lambda b: (b, 0, 0))
    cen = pl.BlockSpec((1, M, 1), lambda b: (b, 0, 0))
    full = lambda s: pl.BlockSpec(s, lambda b: (0, 0))
    return pl.pallas_call(
        _sa2_body,
        grid=(B,),
        in_specs=[
            pl.BlockSpec((1, NP, 128), lambda b: (b, 0, 0)),
            colc, colc, colc, rowc, rowc, rowc, cen, cen, cen,
            full(w1f.shape), full(w1p.shape), full(w2.shape), full(w3.shape),
        ],
        out_specs=pl.BlockSpec((1, M, 256), lambda b: (b, 0, 0)),
        out_shape=jax.ShapeDtypeStruct((B, M, 256), jnp.float32),
    )(feat1, pxc, pyc, pzc, pxr, pyr, pzr, c2x, c2y, c2z, w1f, w1p, w2, w3)


def _sa3fc_body(f_ref, cx_ref, cy_ref, cz_ref, w1f_ref, w1p_ref, w2_ref,
                w3_ref, wf1_ref, b1_ref, wf2_ref, b2_ref, wf3_ref, b3_ref,
                out_ref):
    F = f_ref[0]
    cx, cy, cz = cx_ref[0], cy_ref[0], cz_ref[0]
    w1x = w1p_ref[0:1, :]
    w1y = w1p_ref[1:2, :]
    w1z = w1p_ref[2:3, :]
    A = jnp.dot(F, w1f_ref[...], preferred_element_type=jnp.float32)
    A = A + cx * w1x + cy * w1y + cz * w1z
    x = jnp.maximum(A * _BN_S, 0.0)
    x = jnp.maximum(
        jnp.dot(x, w2_ref[...], preferred_element_type=jnp.float32) * _BN_S,
        0.0)
    x = jnp.maximum(
        jnp.dot(x, w3_ref[...], preferred_element_type=jnp.float32) * _BN_S,
        0.0)
    g = jnp.max(x, axis=0, keepdims=True)
    h = jnp.maximum(
        (jnp.dot(g, wf1_ref[...], preferred_element_type=jnp.float32)
         + b1_ref[...]) * _BN_S, 0.0)
    h = jnp.maximum(
        (jnp.dot(h, wf2_ref[...], preferred_element_type=jnp.float32)
         + b2_ref[...]) * _BN_S, 0.0)
    o = (jnp.dot(h, wf3_ref[...], preferred_element_type=jnp.float32)
         + b3_ref[...])
    out_ref[0] = 1.0 / (1.0 + jnp.exp(-o))


def _sa3fc(feat2, cx, cy, cz, w1f, w1p, w2, w3, wf1, b1, wf2, b2, wf3, b3):
    B, M = feat2.shape[0], feat2.shape[1]
    cen = pl.BlockSpec((1, M, 1), lambda b: (b, 0, 0))
    full = lambda s: pl.BlockSpec(s, lambda b: (0, 0))
    out = pl.pallas_call(
        _sa3fc_body,
        grid=(B,),
        in_specs=[
            pl.BlockSpec((1, M, 256), lambda b: (b, 0, 0)),
            cen, cen, cen,
            full(w1f.shape), full(w1p.shape), full(w2.shape), full(w3.shape),
            full(wf1.shape), full(b1.shape), full(wf2.shape), full(b2.shape),
            full(wf3.shape), full(b3.shape),
        ],
        out_specs=pl.BlockSpec((1, 1, 1), lambda b: (b, 0, 0)),
        out_shape=jax.ShapeDtypeStruct((B, 1, 1), jnp.float32),
    )(feat2, cx, cy, cz, w1f, w1p, w2, w3, wf1, b1, wf2, b2, wf3, b3)
    return out.reshape(B, 1)


def _pad_xyz_rows(w, width):
    t = w[:, :3].T
    return jnp.concatenate(
        [t, jnp.zeros((8 - 3, width), jnp.float32)], axis=0)


def kernel(pc, pc_features, params):
    B, N, _ = pc.shape
    f32 = jnp.float32
    px, py, pz = pc[:, :, 0], pc[:, :, 1], pc[:, :, 2]

    idx1, cx1, cy1, cz1 = _fps(px, py, pz, 128)
    nidx = _bq(px, py, pz, cx1, cy1, cz1, _R2_1, 64)

    feats = jnp.transpose(pc_features, (0, 2, 1))
    table = jnp.concatenate(
        [pc, feats, jnp.zeros((B, N, 9), f32)], axis=-1).reshape(B * N, 16)
    rows = _sc_gather(table, nidx.reshape(-1))

    cen1 = jnp.concatenate(
        [cx1[..., None], cy1[..., None], cz1[..., None],
         jnp.zeros((B, 128, 13), f32)], axis=-1)
    w1a, w2a, w3a = params['sa1']
    w1p = jnp.concatenate(
        [w1a.T, jnp.zeros((16 - 7, 64), f32)], axis=0)
    w1bd = jax.scipy.linalg.block_diag(*([w1p] * 8))
    w2bd = jax.scipy.linalg.block_diag(w2a.T, w2a.T)
    w3bd = jax.scipy.linalg.block_diag(w3a.T, w3a.T)
    feat1 = _mlp1(rows.reshape(B, 1024, 128), cen1, w1p, w1bd, w2bd, w3bd)

    idx2, cx2, cy2, cz2 = _fps(cx1, cy1, cz1, 32)
    w1b, w2b, w3b = params['sa2']
    feat2 = _sa2(
        feat1,
        cx1.reshape(B, 128, 1), cy1.reshape(B, 128, 1), cz1.reshape(B, 128, 1),
        cx1.reshape(B, 1, 128), cy1.reshape(B, 1, 128), cz1.reshape(B, 1, 128),
        cx2.reshape(B, 32, 1), cy2.reshape(B, 32, 1), cz2.reshape(B, 32, 1),
        w1b[:, 3:].T, _pad_xyz_rows(w1b, 128), w2b.T, w3b.T)

    w1c, w2c, w3c = params['sa3']
    wf1, b1, wf2, b2, wf3, b3 = params['fc']
    return _sa3fc(
        feat2,
        cx2.reshape(B, 32, 1), cy2.reshape(B, 32, 1), cz2.reshape(B, 32, 1),
        w1c[:, 3:].T, _pad_xyz_rows(w1c, 256), w2c.T, w3c.T,
        wf1.T, b1.reshape(1, 1024), wf2.T, b2.reshape(1, 1024),
        wf3.T, b3.reshape(1, 1))


# speedup vs baseline: 47.4922x; 1.1772x over previous
import functools

import jax
import jax.numpy as jnp
import numpy as np
from jax import lax
from jax.experimental import pallas as pl
from jax.experimental.pallas import tpu as pltpu
from jax.experimental.pallas import tpu_sc as plsc

_BN_S = float(1.0 / np.sqrt(1.0 + 1e-5))
_R2_1 = float(0.025 * 0.025)
_R2_2 = float(0.05 * 0.05)
_NEG = -1e30


def _fps_body(px_ref, py_ref, pz_ref, idx_ref, cx_ref, cy_ref, cz_ref, *,
              npoint):
    B, N = px_ref.shape
    px = px_ref[...]
    py = py_ref[...]
    pz = pz_ref[...]
    col = lax.broadcasted_iota(jnp.int32, (B, N), 1).astype(jnp.float32)
    colp = lax.broadcasted_iota(jnp.int32, (B, npoint), 1).astype(jnp.float32)

    def body(i, state):
        far, dists, idx_a, cx_a, cy_a, cz_a = state
        sel = col == far
        cx = jnp.sum(jnp.where(sel, px, 0.0), axis=1, keepdims=True)
        cy = jnp.sum(jnp.where(sel, py, 0.0), axis=1, keepdims=True)
        cz = jnp.sum(jnp.where(sel, pz, 0.0), axis=1, keepdims=True)
        hit = colp == i.astype(jnp.float32)
        idx_a = jnp.where(hit, far, idx_a)
        cx_a = jnp.where(hit, cx, cx_a)
        cy_a = jnp.where(hit, cy, cy_a)
        cz_a = jnp.where(hit, cz, cz_a)
        d = (px - cx) ** 2 + (py - cy) ** 2 + (pz - cz) ** 2
        dists = jnp.minimum(dists, d)
        mx = jnp.max(dists, axis=1, keepdims=True)
        far = jnp.min(jnp.where(dists == mx, col, float(N)),
                      axis=1, keepdims=True)
        return far, dists, idx_a, cx_a, cy_a, cz_a

    zp = px[:, :npoint] * 0.0
    state = (
        px[:, :1] * 0.0,
        px * 0.0 + 1e10,
        zp, zp, zp, zp,
    )
    _, _, idx_a, cx_a, cy_a, cz_a = lax.fori_loop(0, npoint, body, state)
    idx_ref[...] = idx_a.astype(jnp.int32)
    cx_ref[...] = cx_a
    cy_ref[...] = cy_a
    cz_ref[...] = cz_a


def _fps(px, py, pz, npoint):
    B, N = px.shape
    f = jnp.float32
    return pl.pallas_call(
        functools.partial(_fps_body, npoint=npoint),
        out_shape=(
            jax.ShapeDtypeStruct((B, npoint), jnp.int32),
            jax.ShapeDtypeStruct((B, npoint), f),
            jax.ShapeDtypeStruct((B, npoint), f),
            jax.ShapeDtypeStruct((B, npoint), f),
        ),
    )(px, py, pz)


def _bq_body(px_ref, py_ref, pz_ref, cx_ref, cy_ref, cz_ref, out_ref, *,
             r2, nsample):
    b = pl.program_id(0)
    npoint = cx_ref.shape[1]
    N = px_ref.shape[-1]
    px = px_ref[0]
    py = py_ref[0]
    pz = pz_ref[0]
    cx = cx_ref[0]
    cy = cy_ref[0]
    cz = cz_ref[0]
    d2 = (cx - px) ** 2 + (cy - py) ** 2 + (cz - pz) ** 2
    col = lax.broadcasted_iota(jnp.int32, (npoint, N), 1).astype(jnp.float32)
    cols = lax.broadcasted_iota(jnp.int32, (npoint, nsample), 1).astype(jnp.float32)
    fN = float(N)
    within = d2 < r2
    cand0 = jnp.where(within, col, fN)
    cnt = jnp.sum(jnp.where(within, 1.0, 0.0), axis=1, keepdims=True)
    cap = jnp.minimum(jnp.max(cnt), float(nsample)).astype(jnp.int32)

    def cond(state):
        s, _, _ = state
        return s < cap

    def body(state):
        s, cand, acc = state
        m = jnp.min(cand, axis=1, keepdims=True)
        acc = jnp.where(cols == s.astype(jnp.float32), m, acc)
        cand = jnp.where(cand == m, fN, cand)
        return s + 1, cand, acc

    _, _, acc = lax.while_loop(
        cond, body, (jnp.int32(0), cand0, d2[:, :nsample] * 0.0 + fN))
    acc = jnp.where(acc == fN, acc[:, 0:1], acc)
    out_ref[0] = acc.astype(jnp.int32) + b * N


def _bq(px, py, pz, cx, cy, cz, r2, nsample):
    B, N = px.shape
    npoint = cx.shape[1]
    pts = pl.BlockSpec((1, 1, N), lambda b: (b, 0, 0))
    cen = pl.BlockSpec((1, npoint, 1), lambda b: (b, 0, 0))
    return pl.pallas_call(
        functools.partial(_bq_body, r2=r2, nsample=nsample),
        grid=(B,),
        in_specs=[pts, pts, pts, cen, cen, cen],
        out_specs=pl.BlockSpec((1, npoint, nsample), lambda b: (b, 0, 0)),
        out_shape=jax.ShapeDtypeStruct((B, npoint, nsample), jnp.int32),
    )(px.reshape(B, 1, N), py.reshape(B, 1, N), pz.reshape(B, 1, N),
      cx.reshape(B, npoint, 1), cy.reshape(B, npoint, 1),
      cz.reshape(B, npoint, 1))


def _sc_gather(table, idx):
    info = plsc.get_sparse_core_info()
    nw = info.num_cores * info.num_subcores
    R = idx.shape[0]
    D = table.shape[1]
    chunk = 128
    n_chunks = R // (nw * chunk)
    idx2 = idx.reshape(nw * n_chunks, chunk)
    mesh = plsc.VectorSubcoreMesh(core_axis_name="c", subcore_axis_name="s")

    @functools.partial(
        pl.kernel,
        mesh=mesh,
        compiler_params=pltpu.CompilerParams(use_tc_tiling_on_sc=False),
        out_type=jax.ShapeDtypeStruct((nw * n_chunks, chunk, D),
                                      jnp.float32),
        scratch_types=[
            pltpu.VMEM((n_chunks, chunk), jnp.int32),
            pltpu.VMEM((n_chunks, chunk, D), jnp.float32),
            pltpu.SemaphoreType.DMA,
        ],
    )
    def k(table_hbm, idx_hbm, out_hbm, idx_v, rows_v, sem):
        wid = lax.axis_index("s") * info.num_cores + lax.axis_index("c")
        base = wid * n_chunks
        pltpu.sync_copy(idx_hbm.at[pl.ds(base, n_chunks)], idx_v)

        def fire(j, carry):
            pltpu.async_copy(table_hbm.at[idx_v.at[j]], rows_v.at[j], sem)
            return carry

        lax.fori_loop(0, n_chunks, fire, 0)

        def drain(j, carry):
            pltpu.make_async_copy(
                table_hbm.at[idx_v.at[j]], rows_v.at[j], sem).wait()
            return carry

        lax.fori_loop(0, n_chunks, drain, 0)
        pltpu.sync_copy(rows_v, out_hbm.at[pl.ds(base, n_chunks)])

    return k(table, idx2).reshape(R, D)


def _mlp1_body(rows_ref, cen_ref, w1_ref, w1bd_ref, w2_ref, w3_ref,
               out_ref):
    C = cen_ref.shape[1]
    x = rows_ref[0]
    cen = cen_ref[0]
    y = jnp.dot(x, w1bd_ref[...],
                preferred_element_type=jnp.float32)
    y = y.reshape(C * 32, 128)
    bc = jnp.dot(cen, w1_ref[...],
                 preferred_element_type=jnp.float32)
    bc = jnp.broadcast_to(bc[:, None, :], (C, 32, 64)).reshape(C * 32, 64)
    bc = jnp.concatenate([bc, bc], axis=1)
    y = jnp.maximum((y - bc) * _BN_S, 0.0)
    y = jnp.maximum(
        jnp.dot(y, w2_ref[...], preferred_element_type=jnp.float32) * _BN_S,
        0.0)
    y = jnp.maximum(
        jnp.dot(y, w3_ref[...], preferred_element_type=jnp.float32) * _BN_S,
        0.0)
    y = y.reshape(C * 64, 128)
    out_ref[0] = jnp.max(y.reshape(C, 64, 128), axis=1)


def _mlp1(rows, cen, w1, w1bd, w2bd, w3bd):
    B, C = cen.shape[0], cen.shape[1]
    full = lambda s: pl.BlockSpec(s, lambda b: (0, 0))
    return pl.pallas_call(
        _mlp1_body,
        grid=(B,),
        in_specs=[
            pl.BlockSpec((1, C * 4, 128), lambda b: (b, 0, 0)),
            pl.BlockSpec((1, C, 8), lambda b: (b, 0, 0)),
            full(w1.shape), full(w1bd.shape), full(w2bd.shape),
            full(w3bd.shape),
        ],
        out_specs=pl.BlockSpec((1, C, 128), lambda b: (b, 0, 0)),
        out_shape=jax.ShapeDtypeStruct((B, C, 128), jnp.float32),
    )(rows, cen, w1, w1bd, w2bd, w3bd)


def _sa2_body(f_ref, pxc_ref, pyc_ref, pzc_ref, pxr_ref, pyr_ref, pzr_ref,
              c2x_ref, c2y_ref, c2z_ref, w1f_ref, w1p_ref, w2_ref, w3_ref,
              out_ref):
    NP = f_ref.shape[1]
    M = c2x_ref.shape[1]
    F = f_ref[0]
    pxc, pyc, pzc = pxc_ref[0], pyc_ref[0], pzc_ref[0]
    pxr, pyr, pzr = pxr_ref[0], pyr_ref[0], pzr_ref[0]
    c2x, c2y, c2z = c2x_ref[0], c2y_ref[0], c2z_ref[0]
    w1x = w1p_ref[0:1, :]
    w1y = w1p_ref[1:2, :]
    w1z = w1p_ref[2:3, :]
    A = jnp.dot(F, w1f_ref[...], preferred_element_type=jnp.float32)
    A = A + pxc * w1x + pyc * w1y + pzc * w1z
    Bc = -(c2x * w1x + c2y * w1y + c2z * w1z)
    x = jnp.maximum((A[None, :, :] + Bc[:, None, :]) * _BN_S, 0.0)
    x = x.reshape(M * NP, 128)
    x = jnp.maximum(
        jnp.dot(x, w2_ref[...], preferred_element_type=jnp.float32) * _BN_S,
        0.0)
    x = jnp.maximum(
        jnp.dot(x, w3_ref[...], preferred_element_type=jnp.float32) * _BN_S,
        0.0)
    d2 = (c2x - pxr) ** 2 + (c2y - pyr) ** 2 + (c2z - pzr) ** 2
    pen = jnp.where(d2 < _R2_2, 0.0, _NEG)
    x = x.reshape(M, NP, 256) + pen[:, :, None]
    out_ref[0] = jnp.max(x, axis=1)


def _sa2(feat1, pxc, pyc, pzc, pxr, pyr, pzr, c2x, c2y, c2z,
         w1f, w1p, w2, w3):
    B, NP = feat1.shape[0], feat1.shape[1]
    M = c2x.shape[1]
    colc = pl.BlockSpec((1, NP, 1), lambda b: (b, 0, 0))
    rowc = pl.BlockSpec((1, 1, NP), lambda b: (b, 0, 0))
    cen = pl.BlockSpec((1, M, 1), lambda b: (b, 0, 0))
    full = lambda s: pl.BlockSpec(s, lambda b: (0, 0))
    return pl.pallas_call(
        _sa2_body,
        grid=(B,),
        in_specs=[
            pl.BlockSpec((1, NP, 128), lambda b: (b, 0, 0)),
            colc, colc, colc, rowc, rowc, rowc, cen, cen, cen,
            full(w1f.shape), full(w1p.shape), full(w2.shape), full(w3.shape),
        ],
        out_specs=pl.BlockSpec((1, M, 256), lambda b: (b, 0, 0)),
        out_shape=jax.ShapeDtypeStruct((B, M, 256), jnp.float32),
    )(feat1, pxc, pyc, pzc, pxr, pyr, pzr, c2x, c2y, c2z, w1f, w1p, w2, w3)


def _sa3fc_body(f_ref, cx_ref, cy_ref, cz_ref, w1f_ref, w1p_ref, w2_ref,
                w3_ref, wf1_ref, b1_ref, wf2_ref, b2_ref, wf3_ref, b3_ref,
                out_ref):
    F = f_ref[0]
    cx, cy, cz = cx_ref[0], cy_ref[0], cz_ref[0]
    w1x = w1p_ref[0:1, :]
    w1y = w1p_ref[1:2, :]
    w1z = w1p_ref[2:3, :]
    A = jnp.dot(F, w1f_ref[...], preferred_element_type=jnp.float32)
    A = A + cx * w1x + cy * w1y + cz * w1z
    x = jnp.maximum(A * _BN_S, 0.0)
    x = jnp.maximum(
        jnp.dot(x, w2_ref[...], preferred_element_type=jnp.float32) * _BN_S,
        0.0)
    x = jnp.maximum(
        jnp.dot(x, w3_ref[...], preferred_element_type=jnp.float32) * _BN_S,
        0.0)
    g = jnp.max(x, axis=0, keepdims=True)
    h = jnp.maximum(
        (jnp.dot(g, wf1_ref[...], preferred_element_type=jnp.float32)
         + b1_ref[...]) * _BN_S, 0.0)
    h = jnp.maximum(
        (jnp.dot(h, wf2_ref[...], preferred_element_type=jnp.float32)
         + b2_ref[...]) * _BN_S, 0.0)
    o = (jnp.dot(h, wf3_ref[...], preferred_element_type=jnp.float32)
         + b3_ref[...])
    out_ref[0] = 1.0 / (1.0 + jnp.exp(-o))


def _sa3fc(feat2, cx, cy, cz, w1f, w1p, w2, w3, wf1, b1, wf2, b2, wf3, b3):
    B, M = feat2.shape[0], feat2.shape[1]
    cen = pl.BlockSpec((1, M, 1), lambda b: (b, 0, 0))
    full = lambda s: pl.BlockSpec(s, lambda b: (0, 0))
    out = pl.pallas_call(
        _sa3fc_body,
        grid=(B,),
        in_specs=[
            pl.BlockSpec((1, M, 256), lambda b: (b, 0, 0)),
            cen, cen, cen,
            full(w1f.shape), full(w1p.shape), full(w2.shape), full(w3.shape),
            full(wf1.shape), full(b1.shape), full(wf2.shape), full(b2.shape),
            full(wf3.shape), full(b3.shape),
        ],
        out_specs=pl.BlockSpec((1, 1, 1), lambda b: (b, 0, 0)),
        out_shape=jax.ShapeDtypeStruct((B, 1, 1), jnp.float32),
    )(feat2, cx, cy, cz, w1f, w1p, w2, w3, wf1, b1, wf2, b2, wf3, b3)
    return out.reshape(B, 1)


def _pad_xyz_rows(w, width):
    t = w[:, :3].T
    return jnp.concatenate(
        [t, jnp.zeros((8 - 3, width), jnp.float32)], axis=0)


def kernel(pc, pc_features, params):
    B, N, _ = pc.shape
    f32 = jnp.float32
    px, py, pz = pc[:, :, 0], pc[:, :, 1], pc[:, :, 2]

    idx1, cx1, cy1, cz1 = _fps(px, py, pz, 128)
    nidx = _bq(px, py, pz, cx1, cy1, cz1, _R2_1, 64)

    feats = jnp.transpose(pc_features, (0, 2, 1))
    table = jnp.concatenate(
        [pc, feats, jnp.zeros((B, N, 1), f32)], axis=-1).reshape(B * N, 8)
    rows = _sc_gather(table, nidx.reshape(-1))

    cen1 = jnp.concatenate(
        [cx1[..., None], cy1[..., None], cz1[..., None],
         jnp.zeros((B, 128, 5), f32)], axis=-1)
    w1a, w2a, w3a = params['sa1']
    w1p = jnp.concatenate(
        [w1a.T, jnp.zeros((1, 64), f32)], axis=0)
    w1bd = jax.scipy.linalg.block_diag(*([w1p] * 16))
    w2bd = jax.scipy.linalg.block_diag(w2a.T, w2a.T)
    w3bd = jax.scipy.linalg.block_diag(w3a.T, w3a.T)
    feat1 = _mlp1(rows.reshape(B, 512, 128), cen1, w1p, w1bd, w2bd, w3bd)

    idx2, cx2, cy2, cz2 = _fps(cx1, cy1, cz1, 32)
    w1b, w2b, w3b = params['sa2']
    feat2 = _sa2(
        feat1,
        cx1.reshape(B, 128, 1), cy1.reshape(B, 128, 1), cz1.reshape(B, 128, 1),
        cx1.reshape(B, 1, 128), cy1.reshape(B, 1, 128), cz1.reshape(B, 1, 128),
        cx2.reshape(B, 32, 1), cy2.reshape(B, 32, 1), cz2.reshape(B, 32, 1),
        w1b[:, 3:].T, _pad_xyz_rows(w1b, 128), w2b.T, w3b.T)

    w1c, w2c, w3c = params['sa3']
    wf1, b1, wf2, b2, wf3, b3 = params['fc']
    return _sa3fc(
        feat2,
        cx2.reshape(B, 32, 1), cy2.reshape(B, 32, 1), cz2.reshape(B, 32, 1),
        w1c[:, 3:].T, _pad_xyz_rows(w1c, 256), w2c.T, w3c.T,
        wf1.T, b1.reshape(1, 1024), wf2.T, b2.reshape(1, 1024),
        wf3.T, b3.reshape(1, 1))


# speedup vs baseline: 50.3887x; 1.0610x over previous
import functools

import jax
import jax.numpy as jnp
import numpy as np
from jax import lax
from jax.experimental import pallas as pl
from jax.experimental.pallas import tpu as pltpu
from jax.experimental.pallas import tpu_sc as plsc

_BN_S = float(1.0 / np.sqrt(1.0 + 1e-5))
_R2_1 = float(0.025 * 0.025)
_R2_2 = float(0.05 * 0.05)
_NEG = -1e30


def _fps_body(px_ref, py_ref, pz_ref, idx_ref, cx_ref, cy_ref, cz_ref, *,
              npoint):
    B, N = px_ref.shape
    px = px_ref[...]
    py = py_ref[...]
    pz = pz_ref[...]
    col = lax.broadcasted_iota(jnp.int32, (B, N), 1).astype(jnp.float32)
    colp = lax.broadcasted_iota(jnp.int32, (B, npoint), 1).astype(jnp.float32)

    def body(i, state):
        far, dists, idx_a, cx_a, cy_a, cz_a = state
        sel = col == far
        cx = jnp.sum(jnp.where(sel, px, 0.0), axis=1, keepdims=True)
        cy = jnp.sum(jnp.where(sel, py, 0.0), axis=1, keepdims=True)
        cz = jnp.sum(jnp.where(sel, pz, 0.0), axis=1, keepdims=True)
        hit = colp == i.astype(jnp.float32)
        idx_a = jnp.where(hit, far, idx_a)
        cx_a = jnp.where(hit, cx, cx_a)
        cy_a = jnp.where(hit, cy, cy_a)
        cz_a = jnp.where(hit, cz, cz_a)
        d = (px - cx) ** 2 + (py - cy) ** 2 + (pz - cz) ** 2
        dists = jnp.minimum(dists, d)
        mx = jnp.max(dists, axis=1, keepdims=True)
        far = jnp.min(jnp.where(dists == mx, col, float(N)),
                      axis=1, keepdims=True)
        return far, dists, idx_a, cx_a, cy_a, cz_a

    zp = px[:, :npoint] * 0.0
    state = (
        px[:, :1] * 0.0,
        px * 0.0 + 1e10,
        zp, zp, zp, zp,
    )
    _, _, idx_a, cx_a, cy_a, cz_a = lax.fori_loop(0, npoint, body, state)
    idx_ref[...] = idx_a.astype(jnp.int32)
    cx_ref[...] = cx_a
    cy_ref[...] = cy_a
    cz_ref[...] = cz_a


def _fps(px, py, pz, npoint):
    B, N = px.shape
    f = jnp.float32
    return pl.pallas_call(
        functools.partial(_fps_body, npoint=npoint),
        out_shape=(
            jax.ShapeDtypeStruct((B, npoint), jnp.int32),
            jax.ShapeDtypeStruct((B, npoint), f),
            jax.ShapeDtypeStruct((B, npoint), f),
            jax.ShapeDtypeStruct((B, npoint), f),
        ),
    )(px, py, pz)


def _bq_body(px_ref, py_ref, pz_ref, cx_ref, cy_ref, cz_ref, out_ref, *,
             r2, nsample):
    b = pl.program_id(0)
    npoint = cx_ref.shape[1]
    N = px_ref.shape[-1]
    px = px_ref[0]
    py = py_ref[0]
    pz = pz_ref[0]
    cx = cx_ref[0]
    cy = cy_ref[0]
    cz = cz_ref[0]
    d2 = (cx - px) ** 2 + (cy - py) ** 2 + (cz - pz) ** 2
    col = lax.broadcasted_iota(jnp.int32, (npoint, N), 1).astype(jnp.float32)
    cols = lax.broadcasted_iota(jnp.int32, (npoint, nsample), 1).astype(jnp.float32)
    fN = float(N)
    within = d2 < r2
    cand0 = jnp.where(within, col, fN)
    cnt = jnp.sum(jnp.where(within, 1.0, 0.0), axis=1, keepdims=True)
    cap = jnp.minimum(jnp.max(cnt), float(nsample)).astype(jnp.int32)

    def cond(state):
        s, _, _ = state
        return s < cap

    def body(state):
        s, cand, acc = state
        m = jnp.min(cand, axis=1, keepdims=True)
        acc = jnp.where(cols == s.astype(jnp.float32), m, acc)
        cand = jnp.where(cand == m, fN, cand)
        return s + 1, cand, acc

    _, _, acc = lax.while_loop(
        cond, body, (jnp.int32(0), cand0, d2[:, :nsample] * 0.0 + fN))
    acc = jnp.where(acc == fN, acc[:, 0:1], acc)
    out_ref[0] = acc.astype(jnp.int32) + b * N


def _bq(px, py, pz, cx, cy, cz, r2, nsample):
    B, N = px.shape
    npoint = cx.shape[1]
    pts = pl.BlockSpec((1, 1, N), lambda b: (b, 0, 0))
    cen = pl.BlockSpec((1, npoint, 1), lambda b: (b, 0, 0))
    return pl.pallas_call(
        functools.partial(_bq_body, r2=r2, nsample=nsample),
        grid=(B,),
        in_specs=[pts, pts, pts, cen, cen, cen],
        out_specs=pl.BlockSpec((1, npoint, nsample), lambda b: (b, 0, 0)),
        out_shape=jax.ShapeDtypeStruct((B, npoint, nsample), jnp.int32),
    )(px.reshape(B, 1, N), py.reshape(B, 1, N), pz.reshape(B, 1, N),
      cx.reshape(B, npoint, 1), cy.reshape(B, npoint, 1),
      cz.reshape(B, npoint, 1))


def _sc_gather(table, idx):
    info = plsc.get_sparse_core_info()
    nw = info.num_cores * info.num_subcores
    R = idx.shape[0]
    D = table.shape[1]
    chunk = 128
    n_chunks = R // (nw * chunk)
    idx2 = idx.reshape(nw * n_chunks, chunk)
    mesh = plsc.VectorSubcoreMesh(core_axis_name="c", subcore_axis_name="s")

    @functools.partial(
        pl.kernel,
        mesh=mesh,
        compiler_params=pltpu.CompilerParams(use_tc_tiling_on_sc=False),
        out_type=jax.ShapeDtypeStruct((nw * n_chunks, chunk, D),
                                      jnp.float32),
        scratch_types=[
            pltpu.VMEM((n_chunks, chunk), jnp.int32),
            pltpu.VMEM((n_chunks, chunk, D), jnp.float32),
            pltpu.SemaphoreType.DMA,
        ],
    )
    def k(table_hbm, idx_hbm, out_hbm, idx_v, rows_v, sem):
        wid = lax.axis_index("s") * info.num_cores + lax.axis_index("c")
        base = wid * n_chunks
        pltpu.sync_copy(idx_hbm.at[pl.ds(base, n_chunks)], idx_v)

        def fire(j, carry):
            pltpu.async_copy(table_hbm.at[idx_v.at[j]], rows_v.at[j], sem)
            return carry

        lax.fori_loop(0, n_chunks, fire, 0)

        def drain(j, carry):
            pltpu.make_async_copy(
                table_hbm.at[idx_v.at[j]], rows_v.at[j], sem).wait()
            return carry

        lax.fori_loop(0, n_chunks, drain, 0)
        pltpu.sync_copy(rows_v, out_hbm.at[pl.ds(base, n_chunks)])

    return k(table, idx2).reshape(R, D)


def _mlp1_body(rows_ref, cen_ref, w1_ref, w1bd_ref, w2_ref, w3_ref,
               out_ref):
    C = cen_ref.shape[1]
    x = rows_ref[0]
    cen = cen_ref[0]
    y = jnp.dot(x, w1bd_ref[...],
                preferred_element_type=jnp.float32)
    y = y.reshape(C * 32, 128)
    bc = jnp.dot(cen, w1_ref[...],
                 preferred_element_type=jnp.float32)
    bc = jnp.broadcast_to(bc[:, None, :], (C, 32, 64)).reshape(C * 32, 64)
    bc = jnp.concatenate([bc, bc], axis=1)
    y = jnp.maximum((y - bc) * _BN_S, 0.0)
    y = jnp.maximum(
        jnp.dot(y, w2_ref[...], preferred_element_type=jnp.float32) * _BN_S,
        0.0)
    y = jnp.maximum(
        jnp.dot(y, w3_ref[...], preferred_element_type=jnp.float32) * _BN_S,
        0.0)
    y = y.reshape(C * 64, 128)
    out_ref[0] = jnp.max(y.reshape(C, 64, 128), axis=1)


def _mlp1(rows, cen, w1, w1bd, w2bd, w3bd):
    B, C = cen.shape[0], cen.shape[1]
    full = lambda s: pl.BlockSpec(s, lambda b: (0, 0))
    return pl.pallas_call(
        _mlp1_body,
        grid=(B,),
        in_specs=[
            pl.BlockSpec((1, C * 4, 128), lambda b: (b, 0, 0)),
            pl.BlockSpec((1, C, 8), lambda b: (b, 0, 0)),
            full(w1.shape), full(w1bd.shape), full(w2bd.shape),
            full(w3bd.shape),
        ],
        out_specs=pl.BlockSpec((1, C, 128), lambda b: (b, 0, 0)),
        out_shape=jax.ShapeDtypeStruct((B, C, 128), jnp.float32),
    )(rows, cen, w1, w1bd, w2bd, w3bd)


def _sa2_body(f_ref, pxc_ref, pyc_ref, pzc_ref, pxr_ref, pyr_ref, pzr_ref,
              c2x_ref, c2y_ref, c2z_ref, w1f_ref, w1p_ref, w2_ref, w3_ref,
              out_ref):
    NP = f_ref.shape[1]
    M = c2x_ref.shape[1]
    F = f_ref[0]
    pxc, pyc, pzc = pxc_ref[0], pyc_ref[0], pzc_ref[0]
    pxr, pyr, pzr = pxr_ref[0], pyr_ref[0], pzr_ref[0]
    c2x, c2y, c2z = c2x_ref[0], c2y_ref[0], c2z_ref[0]
    w1x = w1p_ref[0:1, :]
    w1y = w1p_ref[1:2, :]
    w1z = w1p_ref[2:3, :]
    A = jnp.dot(F, w1f_ref[...], preferred_element_type=jnp.float32)
    A = A + pxc * w1x + pyc * w1y + pzc * w1z
    Bc = -(c2x * w1x + c2y * w1y + c2z * w1z)
    x = jnp.maximum((A[None, :, :] + Bc[:, None, :]) * _BN_S, 0.0)
    x = x.reshape(M * NP, 128)
    x = jnp.maximum(
        jnp.dot(x, w2_ref[...], preferred_element_type=jnp.float32) * _BN_S,
        0.0)
    x = jnp.maximum(
        jnp.dot(x, w3_ref[...], preferred_element_type=jnp.float32) * _BN_S,
        0.0)
    d2 = (c2x - pxr) ** 2 + (c2y - pyr) ** 2 + (c2z - pzr) ** 2
    pen = jnp.where(d2 < _R2_2, 0.0, _NEG)
    x = x.reshape(M, NP, 256) + pen[:, :, None]
    out_ref[0] = jnp.max(x, axis=1)


def _sa2(feat1, pxc, pyc, pzc, pxr, pyr, pzr, c2x, c2y, c2z,
         w1f, w1p, w2, w3):
    B, NP = feat1.shape[0], feat1.shape[1]
    M = c2x.shape[1]
    colc = pl.BlockSpec((1, NP, 1), lambda b: (b, 0, 0))
    rowc = pl.BlockSpec((1, 1, NP), lambda b: (b, 0, 0))
    cen = pl.BlockSpec((1, M, 1), lambda b: (b, 0, 0))
    full = lambda s: pl.BlockSpec(s, lambda b: (0, 0))
    return pl.pallas_call(
        _sa2_body,
        grid=(B,),
        in_specs=[
            pl.BlockSpec((1, NP, 128), lambda b: (b, 0, 0)),
            colc, colc, colc, rowc, rowc, rowc, cen, cen, cen,
            full(w1f.shape), full(w1p.shape), full(w2.shape), full(w3.shape),
        ],
        out_specs=pl.BlockSpec((1, M, 256), lambda b: (b, 0, 0)),
        out_shape=jax.ShapeDtypeStruct((B, M, 256), jnp.float32),
    )(feat1, pxc, pyc, pzc, pxr, pyr, pzr, c2x, c2y, c2z, w1f, w1p, w2, w3)


def _sa3fc_body(f_ref, cx_ref, cy_ref, cz_ref, w1f_ref, w1p_ref, w2_ref,
                w3_ref, wf1_ref, b1_ref, wf2_ref, b2_ref, wf3_ref, b3_ref,
                out_ref):
    B, M = f_ref.shape[0], f_ref.shape[1]
    F = f_ref[...].reshape(B * M, 256)
    cx = cx_ref[...].reshape(B * M, 1)
    cy = cy_ref[...].reshape(B * M, 1)
    cz = cz_ref[...].reshape(B * M, 1)
    w1x = w1p_ref[0:1, :]
    w1y = w1p_ref[1:2, :]
    w1z = w1p_ref[2:3, :]
    A = jnp.dot(F, w1f_ref[...], preferred_element_type=jnp.float32)
    A = A + cx * w1x + cy * w1y + cz * w1z
    x = jnp.maximum(A * _BN_S, 0.0)
    x = jnp.maximum(
        jnp.dot(x, w2_ref[...], preferred_element_type=jnp.float32) * _BN_S,
        0.0)
    x = jnp.maximum(
        jnp.dot(x, w3_ref[...], preferred_element_type=jnp.float32) * _BN_S,
        0.0)
    g = jnp.max(x.reshape(B, M, 1024), axis=1)
    h = jnp.maximum(
        (jnp.dot(g, wf1_ref[...], preferred_element_type=jnp.float32)
         + b1_ref[...]) * _BN_S, 0.0)
    h = jnp.maximum(
        (jnp.dot(h, wf2_ref[...], preferred_element_type=jnp.float32)
         + b2_ref[...]) * _BN_S, 0.0)
    o = (jnp.dot(h, wf3_ref[...], preferred_element_type=jnp.float32)
         + b3_ref[...])
    out_ref[...] = 1.0 / (1.0 + jnp.exp(-o))


def _sa3fc(feat2, cx, cy, cz, w1f, w1p, w2, w3, wf1, b1, wf2, b2, wf3, b3):
    B = feat2.shape[0]
    return pl.pallas_call(
        _sa3fc_body,
        out_shape=jax.ShapeDtypeStruct((B, 1), jnp.float32),
    )(feat2, cx, cy, cz, w1f, w1p, w2, w3, wf1, b1, wf2, b2, wf3, b3)


def _pad_xyz_rows(w, width):
    t = w[:, :3].T
    return jnp.concatenate(
        [t, jnp.zeros((8 - 3, width), jnp.float32)], axis=0)


def kernel(pc, pc_features, params):
    B, N, _ = pc.shape
    f32 = jnp.float32
    px, py, pz = pc[:, :, 0], pc[:, :, 1], pc[:, :, 2]

    idx1, cx1, cy1, cz1 = _fps(px, py, pz, 128)
    nidx = _bq(px, py, pz, cx1, cy1, cz1, _R2_1, 64)

    feats = jnp.transpose(pc_features, (0, 2, 1))
    table = jnp.concatenate(
        [pc, feats, jnp.zeros((B, N, 1), f32)], axis=-1).reshape(B * N, 8)
    rows = _sc_gather(table, nidx.reshape(-1))

    cen1 = jnp.concatenate(
        [cx1[..., None], cy1[..., None], cz1[..., None],
         jnp.zeros((B, 128, 5), f32)], axis=-1)
    w1a, w2a, w3a = params['sa1']
    w1p = jnp.concatenate(
        [w1a.T, jnp.zeros((1, 64), f32)], axis=0)
    w1bd = jax.scipy.linalg.block_diag(*([w1p] * 16))
    w2bd = jax.scipy.linalg.block_diag(w2a.T, w2a.T)
    w3bd = jax.scipy.linalg.block_diag(w3a.T, w3a.T)
    feat1 = _mlp1(rows.reshape(B, 512, 128), cen1, w1p, w1bd, w2bd, w3bd)

    idx2, cx2, cy2, cz2 = _fps(cx1, cy1, cz1, 32)
    w1b, w2b, w3b = params['sa2']
    feat2 = _sa2(
        feat1,
        cx1.reshape(B, 128, 1), cy1.reshape(B, 128, 1), cz1.reshape(B, 128, 1),
        cx1.reshape(B, 1, 128), cy1.reshape(B, 1, 128), cz1.reshape(B, 1, 128),
        cx2.reshape(B, 32, 1), cy2.reshape(B, 32, 1), cz2.reshape(B, 32, 1),
        w1b[:, 3:].T, _pad_xyz_rows(w1b, 128), w2b.T, w3b.T)

    w1c, w2c, w3c = params['sa3']
    wf1, b1, wf2, b2, wf3, b3 = params['fc']
    return _sa3fc(
        feat2,
        cx2.reshape(B, 32, 1), cy2.reshape(B, 32, 1), cz2.reshape(B, 32, 1),
        w1c[:, 3:].T, _pad_xyz_rows(w1c, 256), w2c.T, w3c.T,
        wf1.T, b1.reshape(1, 1024), wf2.T, b2.reshape(1, 1024),
        wf3.T, b3.reshape(1, 1))
